# baseline (speedup 1.0000x reference)
import functools

import jax
import jax.numpy as jnp
from jax import lax
from jax.experimental import pallas as pl
from jax.experimental.pallas import tpu as pltpu
from jax.experimental.pallas import tpu_sc as plsc

N = 10000
E = 320000
C = 128

NC = 2
NS = 16
NW = NC * NS
CHUNK = 128
KCH = 80
EPW = CHUNK * KCH
E_PAD = EPW * NW
N_DUMMY = 240
N_PAD = N + N_DUMMY
STRIPE = N_PAD // NS

_HIGH = lax.Precision.HIGHEST


CONV_BLK = 2000
CONV_GRID = N // CONV_BLK


def _conv_scale_body(feat_ref, w_ref, b_ref, deg_ref, out_ref):
    i = pl.program_id(0)
    base = i * CONV_BLK
    xa = feat_ref[pl.ds(base, CONV_BLK + 16), :]
    xm = xa[7:7 + CONV_BLK, :]
    x = xa[8:8 + CONV_BLK, :]
    xp = xa[9:9 + CONV_BLK, :]
    acc = jnp.dot(xm, w_ref[0], preferred_element_type=jnp.float32,
                  precision=_HIGH)
    acc = acc + jnp.dot(x, w_ref[1], preferred_element_type=jnp.float32,
                        precision=_HIGH)
    acc = acc + jnp.dot(xp, w_ref[2], preferred_element_type=jnp.float32,
                        precision=_HIGH)
    norm = lax.rsqrt(jnp.maximum(deg_ref[pl.ds(base, CONV_BLK), :1], 1.0))
    out_ref[pl.ds(base, CONV_BLK), :] = (acc + b_ref[...]) * norm

    @pl.when(i == CONV_GRID - 1)
    def _():
        out_ref[pl.ds(N, N_DUMMY), :] = jnp.zeros((N_DUMMY, C), jnp.float32)


FIN_BLK = 2000
FIN_GRID = N // FIN_BLK


def _final_body(agg_ref, deg_ref, w_ref, b_ref, feat_ref, out_ref):
    agg = agg_ref[0] + agg_ref[1]
    d = deg_ref[:, :1]
    norm = lax.rsqrt(jnp.maximum(d, 1.0))
    y = jnp.dot(agg, w_ref[...], preferred_element_type=jnp.float32,
                precision=_HIGH)
    out_ref[...] = feat_ref[...] + y * norm + b_ref[...]


def _vmesh():
    return plsc.VectorSubcoreMesh(core_axis_name="c", subcore_axis_name="s")


IBLK = 16
NIB = KCH // IBLK
KCH_D = E_PAD // (NS * CHUNK)
NIB_D = KCH_D // IBLK


SCAT_GRP = 16


def _degree_kernel(src_hbm, dst_hbm, zeros128_hbm, out_hbm,
                   idx_v, ones_v, sem, deg_sh):
    c = lax.axis_index("c")
    s = lax.axis_index("s")
    base = s * STRIPE

    @pl.loop(0, CHUNK)
    def _(j):
        @pl.loop(0, 8)
        def _(k):
            ones_v[j, pl.ds(k * 16, 16)] = jnp.ones((16,), jnp.float32)

    @pl.when(c == 0)
    def _():
        pltpu.sync_copy(src_hbm.at[s], idx_v)

    @pl.when(c == 1)
    def _():
        pltpu.sync_copy(dst_hbm.at[s], idx_v)

    pltpu.sync_copy(zeros128_hbm.at[pl.ds(base, STRIPE)],
                    deg_sh.at[pl.ds(base, STRIPE)])
    plsc.subcore_barrier()

    for g in range(KCH_D // SCAT_GRP):
        handles = [
            pltpu.async_copy(ones_v, deg_sh.at[idx_v.at[g * SCAT_GRP + j]],
                             sem, add=True)
            for j in range(SCAT_GRP)
        ]
        for h in handles:
            h.wait()

    plsc.subcore_barrier()
    pltpu.sync_copy(deg_sh.at[pl.ds(base, STRIPE)],
                    out_hbm.at[c, pl.ds(base, STRIPE)])


NBUF = 2
KPH = KCH // 2


def _msg_kernel(h_hbm, src_hbm, dst_hbm, zeros128_hbm, out_hbm,
                src_v, dst_v, rows0, rows1, sem0, sem1, agg_sh):
    c = lax.axis_index("c")
    s = lax.axis_index("s")
    wid = c * NS + s
    base = s * STRIPE
    pltpu.sync_copy(zeros128_hbm.at[pl.ds(base, STRIPE)],
                    agg_sh.at[pl.ds(base, STRIPE)])
    plsc.subcore_barrier()

    rows = (rows0, rows1)
    sems = (sem0, sem1)
    for p in range(2):
        pltpu.sync_copy(src_hbm.at[wid, pl.ds(p * KPH, KPH)], src_v)
        pltpu.sync_copy(dst_hbm.at[wid, pl.ds(p * KPH, KPH)], dst_v)
        handles = [
            pltpu.async_copy(h_hbm.at[src_v.at[b]], rows[b], sems[b])
            for b in range(NBUF)
        ]
        for b in range(KPH):
            k = b % NBUF
            handles[k].wait()
            pltpu.sync_copy(rows[k], agg_sh.at[dst_v.at[b]], add=True)
            if b + NBUF < KPH:
                handles[k] = pltpu.async_copy(
                    h_hbm.at[src_v.at[b + NBUF]], rows[k], sems[k])

    plsc.subcore_barrier()
    pltpu.sync_copy(agg_sh.at[pl.ds(base, STRIPE)],
                    out_hbm.at[c, pl.ds(base, STRIPE)])


def kernel(feature, edge_index, conv_w, conv_b, gc_w, gc_b):
    f32 = jnp.float32
    w_stack = jnp.transpose(conv_w, (2, 1, 0))
    conv_b2 = conv_b.reshape(1, C)
    gc_b2 = gc_b.reshape(1, C)
    pad_ids = N + (jnp.arange(E_PAD - E, dtype=jnp.int32) % N_DUMMY)
    src_p = jnp.concatenate([edge_index[0], pad_ids]).reshape(NW, KCH, CHUNK)
    dst_p = jnp.concatenate([edge_index[1], pad_ids]).reshape(NW, KCH, CHUNK)
    zeros128 = jnp.zeros((N_PAD, C), f32)

    feat_pad = jnp.pad(feature, ((8, 8), (0, 0)))

    src_d = src_p.reshape(NS, KCH_D, CHUNK)
    dst_d = dst_p.reshape(NS, KCH_D, CHUNK)
    degs = pl.kernel(
        _degree_kernel,
        out_type=jax.ShapeDtypeStruct((NC, N_PAD, C), f32),
        mesh=_vmesh(),
        scratch_types=[
            pltpu.VMEM((KCH_D, CHUNK), jnp.int32),
            pltpu.VMEM((CHUNK, C), f32),
            pltpu.SemaphoreType.DMA,
            pltpu.VMEM_SHARED((N_PAD, C), f32),
        ],
    )(src_d, dst_d, zeros128)

    h_scaled = pl.pallas_call(
        _conv_scale_body,
        grid=(CONV_GRID,),
        in_specs=[
            pl.BlockSpec((N + 16, C), lambda i: (0, 0)),
            pl.BlockSpec((3, C, C), lambda i: (0, 0, 0)),
            pl.BlockSpec((1, C), lambda i: (0, 0)),
            pl.BlockSpec((N_PAD, C), lambda i: (0, 0)),
        ],
        out_specs=pl.BlockSpec((N_PAD, C), lambda i: (0, 0)),
        out_shape=jax.ShapeDtypeStruct((N_PAD, C), f32),
    )(feat_pad, w_stack, conv_b2, degs[0])

    agg = pl.kernel(
        _msg_kernel,
        out_type=jax.ShapeDtypeStruct((NC, N_PAD, C), f32),
        mesh=_vmesh(),
        scratch_types=[
            pltpu.VMEM((KPH, CHUNK), jnp.int32),
            pltpu.VMEM((KPH, CHUNK), jnp.int32),
            pltpu.VMEM((CHUNK, C), f32),
            pltpu.VMEM((CHUNK, C), f32),
            pltpu.SemaphoreType.DMA,
            pltpu.SemaphoreType.DMA,
            pltpu.VMEM_SHARED((N_PAD, C), f32),
        ],
    )(h_scaled, src_p, dst_p, zeros128)

    out = pl.pallas_call(
        _final_body,
        grid=(FIN_GRID,),
        in_specs=[
            pl.BlockSpec((NC, FIN_BLK, C), lambda i: (0, i, 0)),
            pl.BlockSpec((FIN_BLK, C), lambda i: (i, 0)),
            pl.BlockSpec((C, C), lambda i: (0, 0)),
            pl.BlockSpec((1, C), lambda i: (0, 0)),
            pl.BlockSpec((FIN_BLK, C), lambda i: (i, 0)),
        ],
        out_specs=pl.BlockSpec((FIN_BLK, C), lambda i: (i, 0)),
        out_shape=jax.ShapeDtypeStruct((N, C), f32),
    )(agg, degs[1], gc_w, gc_b2, feature)
    return out


# speedup vs baseline: 1.0638x; 1.0638x over previous
import functools

import jax
import jax.numpy as jnp
from jax import lax
from jax.experimental import pallas as pl
from jax.experimental.pallas import tpu as pltpu
from jax.experimental.pallas import tpu_sc as plsc

N = 10000
E = 320000
C = 128

NC = 2
NS = 16
NW = NC * NS
CHUNK = 128
KCH = 80
EPW = CHUNK * KCH
E_PAD = EPW * NW
N_DUMMY = 240
N_PAD = N + N_DUMMY
STRIPE = N_PAD // NS

_HIGH = lax.Precision.HIGHEST


CONV_BLK = 2000
CONV_GRID = N // CONV_BLK


def _conv_body(feat_ref, w_ref, b_ref, out_ref):
    i = pl.program_id(0)
    base = i * CONV_BLK
    xa = feat_ref[pl.ds(base, CONV_BLK + 16), :]
    xm = xa[7:7 + CONV_BLK, :]
    x = xa[8:8 + CONV_BLK, :]
    xp = xa[9:9 + CONV_BLK, :]
    acc = jnp.dot(xm, w_ref[0], preferred_element_type=jnp.float32,
                  precision=_HIGH)
    acc = acc + jnp.dot(x, w_ref[1], preferred_element_type=jnp.float32,
                        precision=_HIGH)
    acc = acc + jnp.dot(xp, w_ref[2], preferred_element_type=jnp.float32,
                        precision=_HIGH)
    out_ref[...] = acc + b_ref[...]


def _scale_body(h_ref, deg_ref, out_ref):
    d = deg_ref[:, :1]
    norm = lax.rsqrt(jnp.maximum(d, 1.0))
    out_ref[0:N, :] = h_ref[...] * norm[:N]
    out_ref[N:N_PAD, :] = jnp.zeros((N_DUMMY, C), jnp.float32)


FIN_BLK = 2000
FIN_GRID = N // FIN_BLK


def _final_body(agg_ref, deg_ref, w_ref, b_ref, feat_ref, out_ref):
    agg = agg_ref[0] + agg_ref[1]
    d = deg_ref[:, :1]
    norm = lax.rsqrt(jnp.maximum(d, 1.0))
    y = jnp.dot(agg, w_ref[...], preferred_element_type=jnp.float32,
                precision=_HIGH)
    out_ref[...] = feat_ref[...] + y * norm + b_ref[...]


def _vmesh():
    return plsc.VectorSubcoreMesh(core_axis_name="c", subcore_axis_name="s")


IBLK = 16
NIB = KCH // IBLK
KCH_D = E_PAD // (NS * CHUNK)
NIB_D = KCH_D // IBLK


SCAT_GRP = 8


def _degree_kernel(src_hbm, dst_hbm, zeros128_hbm, out_hbm,
                   idx_v, ones_v, sem, deg_sh):
    c = lax.axis_index("c")
    s = lax.axis_index("s")
    base = s * STRIPE

    @pl.loop(0, CHUNK)
    def _(j):
        @pl.loop(0, 8)
        def _(k):
            ones_v[j, pl.ds(k * 16, 16)] = jnp.ones((16,), jnp.float32)

    @pl.when(c == 0)
    def _():
        pltpu.sync_copy(src_hbm.at[s], idx_v)

    @pl.when(c == 1)
    def _():
        pltpu.sync_copy(dst_hbm.at[s], idx_v)

    pltpu.sync_copy(zeros128_hbm.at[pl.ds(base, STRIPE)],
                    deg_sh.at[pl.ds(base, STRIPE)])
    plsc.subcore_barrier()

    for g in range(KCH_D // SCAT_GRP):
        handles = [
            pltpu.async_copy(ones_v, deg_sh.at[idx_v.at[g * SCAT_GRP + j]],
                             sem, add=True)
            for j in range(SCAT_GRP)
        ]
        for h in handles:
            h.wait()

    plsc.subcore_barrier()
    pltpu.sync_copy(deg_sh.at[pl.ds(base, STRIPE)],
                    out_hbm.at[c, pl.ds(base, STRIPE)])


NBUF = 4
MCH = 64
MTOT = EPW // MCH
MPH = 4
MKPH = MTOT // MPH


def _msg_kernel(h_hbm, src_hbm, dst_hbm, zeros128_hbm, out_hbm,
                src_v, dst_v, rows0, rows1, rows2, rows3,
                sem0, sem1, sem2, sem3, agg_sh):
    c = lax.axis_index("c")
    s = lax.axis_index("s")
    wid = c * NS + s
    base = s * STRIPE
    pltpu.sync_copy(zeros128_hbm.at[pl.ds(base, STRIPE)],
                    agg_sh.at[pl.ds(base, STRIPE)])
    plsc.subcore_barrier()

    rows = (rows0, rows1, rows2, rows3)
    sems = (sem0, sem1, sem2, sem3)
    for p in range(MPH):
        pltpu.sync_copy(src_hbm.at[wid, pl.ds(p * MKPH, MKPH)], src_v)
        pltpu.sync_copy(dst_hbm.at[wid, pl.ds(p * MKPH, MKPH)], dst_v)
        handles = [
            pltpu.async_copy(h_hbm.at[src_v.at[b]], rows[b], sems[b])
            for b in range(NBUF)
        ]
        for b in range(MKPH):
            k = b % NBUF
            handles[k].wait()
            pltpu.sync_copy(rows[k], agg_sh.at[dst_v.at[b]], add=True)
            if b + NBUF < MKPH:
                handles[k] = pltpu.async_copy(
                    h_hbm.at[src_v.at[b + NBUF]], rows[k], sems[k])

    plsc.subcore_barrier()
    pltpu.sync_copy(agg_sh.at[pl.ds(base, STRIPE)],
                    out_hbm.at[c, pl.ds(base, STRIPE)])


def kernel(feature, edge_index, conv_w, conv_b, gc_w, gc_b):
    f32 = jnp.float32
    w_stack = jnp.transpose(conv_w, (2, 1, 0))
    conv_b2 = conv_b.reshape(1, C)
    gc_b2 = gc_b.reshape(1, C)
    pad_ids = N + (jnp.arange(E_PAD - E, dtype=jnp.int32) % N_DUMMY)
    src_p = jnp.concatenate([edge_index[0], pad_ids]).reshape(NW, KCH, CHUNK)
    dst_p = jnp.concatenate([edge_index[1], pad_ids]).reshape(NW, KCH, CHUNK)
    zeros128 = jnp.zeros((N_PAD, C), f32)

    feat_pad = jnp.pad(feature, ((8, 8), (0, 0)))
    h_conv = pl.pallas_call(
        _conv_body,
        grid=(CONV_GRID,),
        in_specs=[
            pl.BlockSpec((N + 16, C), lambda i: (0, 0)),
            pl.BlockSpec((3, C, C), lambda i: (0, 0, 0)),
            pl.BlockSpec((1, C), lambda i: (0, 0)),
        ],
        out_specs=pl.BlockSpec((CONV_BLK, C), lambda i: (i, 0)),
        out_shape=jax.ShapeDtypeStruct((N, C), f32),
    )(feat_pad, w_stack, conv_b2)

    src_d = src_p.reshape(NS, KCH_D, CHUNK)
    dst_d = dst_p.reshape(NS, KCH_D, CHUNK)
    src_m = src_p.reshape(NW, MTOT, MCH)
    dst_m = dst_p.reshape(NW, MTOT, MCH)
    degs = pl.kernel(
        _degree_kernel,
        out_type=jax.ShapeDtypeStruct((NC, N_PAD, C), f32),
        mesh=_vmesh(),
        scratch_types=[
            pltpu.VMEM((KCH_D, CHUNK), jnp.int32),
            pltpu.VMEM((CHUNK, C), f32),
            pltpu.SemaphoreType.DMA,
            pltpu.VMEM_SHARED((N_PAD, C), f32),
        ],
    )(src_d, dst_d, zeros128)

    h_scaled = pl.pallas_call(
        _scale_body,
        out_shape=jax.ShapeDtypeStruct((N_PAD, C), f32),
    )(h_conv, degs[0])

    agg = pl.kernel(
        _msg_kernel,
        out_type=jax.ShapeDtypeStruct((NC, N_PAD, C), f32),
        mesh=_vmesh(),
        scratch_types=[
            pltpu.VMEM((MKPH, MCH), jnp.int32),
            pltpu.VMEM((MKPH, MCH), jnp.int32),
            pltpu.VMEM((MCH, C), f32),
            pltpu.VMEM((MCH, C), f32),
            pltpu.VMEM((MCH, C), f32),
            pltpu.VMEM((MCH, C), f32),
            pltpu.SemaphoreType.DMA,
            pltpu.SemaphoreType.DMA,
            pltpu.SemaphoreType.DMA,
            pltpu.SemaphoreType.DMA,
            pltpu.VMEM_SHARED((N_PAD, C), f32),
        ],
    )(h_scaled, src_m, dst_m, zeros128)

    out = pl.pallas_call(
        _final_body,
        grid=(FIN_GRID,),
        in_specs=[
            pl.BlockSpec((NC, FIN_BLK, C), lambda i: (0, i, 0)),
            pl.BlockSpec((FIN_BLK, C), lambda i: (i, 0)),
            pl.BlockSpec((C, C), lambda i: (0, 0)),
            pl.BlockSpec((1, C), lambda i: (0, 0)),
            pl.BlockSpec((FIN_BLK, C), lambda i: (i, 0)),
        ],
        out_specs=pl.BlockSpec((FIN_BLK, C), lambda i: (i, 0)),
        out_shape=jax.ShapeDtypeStruct((N, C), f32),
    )(agg, degs[1], gc_w, gc_b2, feature)
    return out
